# Initial kernel scaffold; baseline (speedup 1.0000x reference)
import jax
import jax.numpy as jnp
from jax.experimental import pallas as pl


def kernel(x, y):
    raise NotImplementedError("write your pallas kernel here")


# speedup vs baseline: 1.1507x; 1.1507x over previous
import functools

import jax
import jax.numpy as jnp
from jax import lax
from jax.experimental import pallas as pl
from jax.experimental.pallas import tpu as pltpu
from jax.experimental.pallas import tpu_sc as plsc

_SCALE = 30.0
_MARGIN = 0.4
_B = 1024
_C = 100000


_NC = 2
_NS = 16
_NW = _NC * _NS
_RPW = _B // _NW
_GRAN = 16
_XROWS = _B * _C // _GRAN

def _sc_gather_body(x_hbm, y_hbm, out_hbm, y_v, f_v, t_v, sem):
    wid = lax.axis_index("s") * _NC + lax.axis_index("c")
    base = wid * _RPW
    pltpu.sync_copy(y_hbm.at[pl.ds(base, _RPW)], y_v)
    iota = lax.broadcasted_iota(jnp.int32, (16,), 0)
    for g in range(_RPW // 16):
        yv = y_v[pl.ds(g * 16, 16)]
        f_v[pl.ds(g * 16, 16)] = (base + g * 16 + iota) * _C + yv
    pltpu.async_copy(x_hbm.at[f_v], t_v, sem).wait()
    pltpu.sync_copy(t_v, out_hbm.at[pl.ds(base, _RPW)])


@functools.cache
def _sc_gather_targets():
    mesh = plsc.VectorSubcoreMesh(
        core_axis_name="c", subcore_axis_name="s",
        num_cores=_NC, num_subcores=_NS)
    return pl.kernel(
        _sc_gather_body,
        out_type=jax.ShapeDtypeStruct((_B,), jnp.float32),
        mesh=mesh,
        scratch_types=[
            pltpu.VMEM((_RPW,), jnp.int32),
            pltpu.VMEM((_RPW,), jnp.int32),
            pltpu.VMEM((_RPW,), jnp.float32),
            pltpu.SemaphoreType.DMA,
        ],
    )


_BC = 2048
_K = (_C + _BC - 1) // _BC


def _tc_lse_body(t_ref, x_ref, out_ref, m_ref, s_ref):
    k = pl.program_id(0)
    nk = pl.num_programs(0)

    @pl.when(k == 0)
    def _init():
        m_ref[...] = jnp.full_like(m_ref, -1e30)
        s_ref[...] = jnp.zeros_like(s_ref)

    l = x_ref[...] * _SCALE
    col = k * _BC + lax.broadcasted_iota(jnp.int32, l.shape, 1)
    l = jnp.where(col < _C, l, -1e30)
    m_old = m_ref[...]
    m_new = jnp.maximum(m_old, jnp.max(l, axis=1, keepdims=True))
    s_ref[...] = (s_ref[...] * jnp.exp(m_old - m_new)
                  + jnp.sum(jnp.exp(l - m_new), axis=1, keepdims=True))
    m_ref[...] = m_new

    @pl.when(k == nk - 1)
    def _finalize():
        m = m_ref[...]
        lt = t_ref[...] * _SCALE
        s_mod = (s_ref[...] - jnp.exp(lt - m)
                 + jnp.exp(lt - _SCALE * _MARGIN - m))
        row_loss = m + jnp.log(s_mod) - (lt - _SCALE * _MARGIN)
        out_ref[0, 0] = jnp.sum(row_loss) * (1.0 / _B)


_tc_lse = pl.pallas_call(
    _tc_lse_body,
    grid=(_K,),
    in_specs=[
        pl.BlockSpec((_B, 1), lambda k: (0, 0)),
        pl.BlockSpec((_B, _BC), lambda k: (0, k)),
    ],
    out_specs=pl.BlockSpec(memory_space=pltpu.SMEM),
    out_shape=jax.ShapeDtypeStruct((1, 1), jnp.float32),
    scratch_shapes=[
        pltpu.VMEM((_B, 1), jnp.float32),
        pltpu.VMEM((_B, 1), jnp.float32),
    ],
)


def kernel(x, y):
    y32 = y.astype(jnp.int32)
    t = _sc_gather_targets()(x.reshape(_B * _C), y32)
    loss = _tc_lse(t.reshape(_B, 1), x)
    return loss[0, 0]


# speedup vs baseline: 2.4137x; 2.0976x over previous
import jax
import jax.numpy as jnp
from jax import lax
from jax.experimental import pallas as pl
from jax.experimental.pallas import tpu as pltpu

_SCALE = 30.0
_MARGIN = 0.4
_B = 1024
_C = 100000
_BC = 2048
_K = (_C + _BC - 1) // _BC


def _tc_body(y_ref, x_ref, out_ref, m_ref, s_ref, t_ref):
    k = pl.program_id(0)
    nk = pl.num_programs(0)

    @pl.when(k == 0)
    def _init():
        m_ref[...] = jnp.full_like(m_ref, -1e30)
        s_ref[...] = jnp.zeros_like(s_ref)
        t_ref[...] = jnp.zeros_like(t_ref)

    xb = x_ref[...]
    col = k * _BC + lax.broadcasted_iota(jnp.int32, xb.shape, 1)
    l = jnp.where(col < _C, xb * _SCALE, -1e30)
    t_ref[...] += jnp.sum(jnp.where(col == y_ref[...], xb, 0.0), axis=1,
                          keepdims=True)
    m_old = m_ref[...]
    m_new = jnp.maximum(m_old, jnp.max(l, axis=1, keepdims=True))
    s_ref[...] = (s_ref[...] * jnp.exp(m_old - m_new)
                  + jnp.sum(jnp.exp(l - m_new), axis=1, keepdims=True))
    m_ref[...] = m_new

    @pl.when(k == nk - 1)
    def _finalize():
        m = m_ref[...]
        lt = t_ref[...] * _SCALE
        s_mod = (s_ref[...] - jnp.exp(lt - m)
                 + jnp.exp(lt - _SCALE * _MARGIN - m))
        row_loss = m + jnp.log(s_mod) - (lt - _SCALE * _MARGIN)
        out_ref[0, 0] = jnp.sum(row_loss) * (1.0 / _B)


_tc_lse = pl.pallas_call(
    _tc_body,
    grid=(_K,),
    in_specs=[
        pl.BlockSpec((_B, 1), lambda k: (0, 0)),
        pl.BlockSpec((_B, _BC), lambda k: (0, k)),
    ],
    out_specs=pl.BlockSpec(memory_space=pltpu.SMEM),
    out_shape=jax.ShapeDtypeStruct((1, 1), jnp.float32),
    scratch_shapes=[
        pltpu.VMEM((_B, 1), jnp.float32),
        pltpu.VMEM((_B, 1), jnp.float32),
        pltpu.VMEM((_B, 1), jnp.float32),
    ],
)


def kernel(x, y):
    y32 = y.astype(jnp.int32).reshape(_B, 1)
    loss = _tc_lse(y32, x)
    return loss[0, 0]


# speedup vs baseline: 2.4139x; 1.0001x over previous
import jax
import jax.numpy as jnp
from jax import lax
from jax.experimental import pallas as pl
from jax.experimental.pallas import tpu as pltpu

_SCALE = 30.0
_MARGIN = 0.4
_B = 1024
_C = 100000
_BC = 3584
_K = (_C + _BC - 1) // _BC


def _tc_body(y_ref, x_ref, out_ref, m_ref, s_ref, t_ref):
    k = pl.program_id(0)
    nk = pl.num_programs(0)

    @pl.when(k == 0)
    def _init():
        m_ref[...] = jnp.full_like(m_ref, -1e30)
        s_ref[...] = jnp.zeros_like(s_ref)
        t_ref[...] = jnp.zeros_like(t_ref)

    xb = x_ref[...]
    col = k * _BC + lax.broadcasted_iota(jnp.int32, xb.shape, 1)
    l = jnp.where(col < _C, xb * _SCALE, -1e30)
    t_ref[...] += jnp.sum(jnp.where(col == y_ref[...], xb, 0.0), axis=1,
                          keepdims=True)
    m_old = m_ref[...]
    m_new = jnp.maximum(m_old, jnp.max(l, axis=1, keepdims=True))
    s_ref[...] = (s_ref[...] * jnp.exp(m_old - m_new)
                  + jnp.sum(jnp.exp(l - m_new), axis=1, keepdims=True))
    m_ref[...] = m_new

    @pl.when(k == nk - 1)
    def _finalize():
        m = m_ref[...]
        lt = t_ref[...] * _SCALE
        s_mod = (s_ref[...] - jnp.exp(lt - m)
                 + jnp.exp(lt - _SCALE * _MARGIN - m))
        row_loss = m + jnp.log(s_mod) - (lt - _SCALE * _MARGIN)
        out_ref[0, 0] = jnp.sum(row_loss) * (1.0 / _B)


_tc_lse = pl.pallas_call(
    _tc_body,
    grid=(_K,),
    in_specs=[
        pl.BlockSpec((_B, 1), lambda k: (0, 0)),
        pl.BlockSpec((_B, _BC), lambda k: (0, k)),
    ],
    out_specs=pl.BlockSpec(memory_space=pltpu.SMEM),
    out_shape=jax.ShapeDtypeStruct((1, 1), jnp.float32),
    scratch_shapes=[
        pltpu.VMEM((_B, 1), jnp.float32),
        pltpu.VMEM((_B, 1), jnp.float32),
        pltpu.VMEM((_B, 1), jnp.float32),
    ],
)


def kernel(x, y):
    y32 = y.astype(jnp.int32).reshape(_B, 1)
    loss = _tc_lse(y32, x)
    return loss[0, 0]


# speedup vs baseline: 2.4604x; 1.0193x over previous
import jax
import jax.numpy as jnp
from jax import lax
from jax.experimental import pallas as pl
from jax.experimental.pallas import tpu as pltpu

_SCALE = 30.0
_MARGIN = 0.4
_B = 1024
_C = 100000
_NSPLIT = 4
_BCS = 896
_BCT = _NSPLIT * _BCS
_K = (_C + _BCT - 1) // _BCT


def _tc_body(y_ref, x0, x1, x2, x3, out_ref, m_ref, s_ref, t_ref):
    k = pl.program_id(0)
    nk = pl.num_programs(0)

    @pl.when(k == 0)
    def _init():
        m_ref[...] = jnp.full_like(m_ref, -1e30)
        s_ref[...] = jnp.zeros_like(s_ref)
        t_ref[...] = jnp.zeros_like(t_ref)

    y = y_ref[...]
    t_acc = jnp.zeros((_B, 1), jnp.float32)
    m_old = m_ref[...]
    m_new = m_old
    parts = []
    for j, xr in enumerate((x0, x1, x2, x3)):
        xb = xr[...]
        col = (k * _NSPLIT + j) * _BCS + lax.broadcasted_iota(
            jnp.int32, xb.shape, 1)
        l = jnp.where(col < _C, xb * _SCALE, -1e30)
        parts.append(l)
        t_acc += jnp.sum(jnp.where(col == y, xb, 0.0), axis=1, keepdims=True)
        m_new = jnp.maximum(m_new, jnp.max(l, axis=1, keepdims=True))
    t_ref[...] += t_acc
    s_acc = s_ref[...] * jnp.exp(m_old - m_new)
    for l in parts:
        s_acc += jnp.sum(jnp.exp(l - m_new), axis=1, keepdims=True)
    s_ref[...] = s_acc
    m_ref[...] = m_new

    @pl.when(k == nk - 1)
    def _finalize():
        m = m_ref[...]
        lt = t_ref[...] * _SCALE
        s_mod = (s_ref[...] - jnp.exp(lt - m)
                 + jnp.exp(lt - _SCALE * _MARGIN - m))
        row_loss = m + jnp.log(s_mod) - (lt - _SCALE * _MARGIN)
        out_ref[0, 0] = jnp.sum(row_loss) * (1.0 / _B)


def _x_spec(j):
    return pl.BlockSpec((_B, _BCS), lambda k, j=j: (0, k * _NSPLIT + j))


_tc_lse = pl.pallas_call(
    _tc_body,
    grid=(_K,),
    in_specs=[pl.BlockSpec((_B, 1), lambda k: (0, 0))]
    + [_x_spec(j) for j in range(_NSPLIT)],
    out_specs=pl.BlockSpec(memory_space=pltpu.SMEM),
    out_shape=jax.ShapeDtypeStruct((1, 1), jnp.float32),
    scratch_shapes=[
        pltpu.VMEM((_B, 1), jnp.float32),
        pltpu.VMEM((_B, 1), jnp.float32),
        pltpu.VMEM((_B, 1), jnp.float32),
    ],
)


def kernel(x, y):
    y32 = y.astype(jnp.int32).reshape(_B, 1)
    loss = _tc_lse(y32, x, x, x, x)
    return loss[0, 0]


# speedup vs baseline: 2.5069x; 1.0189x over previous
import functools

import jax
import jax.numpy as jnp
from jax import lax
from jax.experimental import pallas as pl
from jax.experimental.pallas import tpu as pltpu
from jax.experimental.pallas import tpu_sc as plsc

_SCALE = 30.0
_MARGIN = 0.4
_B = 1024
_C = 100000
_LN2 = 0.6931471805599453
_K2 = _SCALE / _LN2


_NC = 2
_NS = 16
_NW = _NC * _NS
_RPW = _B // _NW
_GRAN = 128


def _sc_gather_body(x_hbm, y_hbm, out_hbm, y_v, tiles_v, stage_v, sem):
    wid = lax.axis_index("s") * _NC + lax.axis_index("c")
    base = wid * _RPW
    pltpu.sync_copy(y_hbm.at[pl.ds(base, _RPW)], y_v)
    iota = lax.broadcasted_iota(jnp.int32, (16,), 0)
    copies = []
    for j in range(_RPW):
        g, lane = divmod(j, 16)
        yv = y_v[pl.ds(g * 16, 16)]
        onehot = (iota == lane).astype(jnp.float32)
        yj = jnp.sum(yv.astype(jnp.float32) * onehot).astype(jnp.int32)
        c0 = pl.multiple_of(yj - lax.rem(yj, _GRAN), _GRAN)
        r0 = pl.multiple_of(base + 8 * (j // 8), 8)
        copies.append(
            pltpu.async_copy(
                x_hbm.at[pl.ds(r0, 8), pl.ds(c0, _GRAN)],
                tiles_v.at[j], sem))
    for c in copies:
        c.wait()
    for j in range(_RPW):
        r = j % 8
        for q in range(_GRAN // 16):
            stage_v[j, pl.ds(q * 16, 16)] = tiles_v[j, r, pl.ds(q * 16, 16)]
    pltpu.sync_copy(stage_v, out_hbm.at[pl.ds(base, _RPW)])


@functools.cache
def _sc_gather_targets():
    mesh = plsc.VectorSubcoreMesh(
        core_axis_name="c", subcore_axis_name="s",
        num_cores=_NC, num_subcores=_NS)
    return pl.kernel(
        _sc_gather_body,
        out_type=jax.ShapeDtypeStruct((_B, _GRAN), jnp.float32),
        mesh=mesh,
        scratch_types=[
            pltpu.VMEM((_RPW,), jnp.int32),
            pltpu.VMEM((_RPW, 8, _GRAN), jnp.float32),
            pltpu.VMEM((_RPW, _GRAN), jnp.float32),
            pltpu.SemaphoreType.DMA,
        ],
        compiler_params=pltpu.CompilerParams(use_tc_tiling_on_sc=True),
    )


_BC = 3584
_K = (_C + _BC - 1) // _BC
_NL = 128
_NV = _BC // _NL


def _tc_body(y_ref, x_ref, m_out, s_out, t_out, m_ref, s_ref, t_ref):
    k = pl.program_id(0)
    nk = pl.num_programs(0)

    @pl.when(k == 0)
    def _init():
        m_ref[...] = jnp.full_like(m_ref, -1e30)
        s_ref[...] = jnp.zeros_like(s_ref)
        t_ref[...] = jnp.zeros_like(t_ref)

    def step(xb, col):
        y2 = y_ref[...]
        t_ref[...] += jnp.sum(jnp.where(col == y2, xb, 0.0), axis=1,
                              keepdims=True)
        m_old = m_ref[...]
        m_new = jnp.maximum(m_old, jnp.max(xb, axis=1, keepdims=True))
        s_ref[...] = (s_ref[...] * jnp.exp2((m_old - m_new) * _K2)
                      + jnp.sum(jnp.exp2((xb - m_new) * _K2), axis=1,
                                keepdims=True))
        m_ref[...] = m_new

    def colidx(k):
        return k * _BC + lax.broadcasted_iota(jnp.int32, (_B, _BC), 1)

    @pl.when(k < nk - 1)
    def _hot():
        step(x_ref[...], colidx(k))

    @pl.when(k == nk - 1)
    def _tail():
        col = colidx(k)
        step(jnp.where(col < _C, x_ref[...], -1e30), col)
        m_out[...] = m_ref[...]
        s_out[...] = s_ref[...]
        t_out[...] = t_ref[...]


_tc_lse = pl.pallas_call(
    _tc_body,
    grid=(_K,),
    in_specs=[
        pl.BlockSpec((_B, 1), lambda k: (0, 0)),
        pl.BlockSpec((_B, _BC), lambda k: (0, k)),
    ],
    out_specs=[
        pl.BlockSpec((_B, 1), lambda k: (0, 0)),
        pl.BlockSpec((_B, 1), lambda k: (0, 0)),
        pl.BlockSpec((_B, 1), lambda k: (0, 0)),
    ],
    out_shape=[
        jax.ShapeDtypeStruct((_B, 1), jnp.float32),
        jax.ShapeDtypeStruct((_B, 1), jnp.float32),
        jax.ShapeDtypeStruct((_B, 1), jnp.float32),
    ],
    scratch_shapes=[
        pltpu.VMEM((_B, 1), jnp.float32),
        pltpu.VMEM((_B, 1), jnp.float32),
        pltpu.VMEM((_B, 1), jnp.float32),
    ],
)


def _combine_body(m_ref, s_ref, t_ref, out_ref):
    m = m_ref[...]
    s = s_ref[...]
    t = t_ref[...]
    zt = (t - m) * _K2
    s_mod = s - jnp.exp2(zt) + jnp.exp2(zt - _SCALE * _MARGIN / _LN2)
    row_loss = (_LN2 * _K2 * m + jnp.log(s_mod)
                - _SCALE * t + _SCALE * _MARGIN)
    out_ref[0, 0] = jnp.sum(row_loss) * (1.0 / _B)


_combine = pl.pallas_call(
    _combine_body,
    out_specs=pl.BlockSpec(memory_space=pltpu.SMEM),
    out_shape=jax.ShapeDtypeStruct((1, 1), jnp.float32),
)


def kernel(x, y):
    y32 = y.astype(jnp.int32)
    m, s, t = _tc_lse(y32.reshape(_B, 1), x)
    loss = _combine(m, s, t)
    return loss[0, 0]
